# baseline (speedup 1.0000x reference)
import jax
import jax.numpy as jnp
from jax import lax
from jax.experimental import pallas as pl
from jax.experimental.pallas import tpu as pltpu
from jax.experimental.pallas import tpu_sc as plsc

N = 100000
C = 128
NC, NS = 2, 16
NW = NC * NS
R = N // NW
CH = 224
NCHUNK = -(-R // CH)
LASTCH = R - (NCHUNK - 1) * CH
G = CH // 16
LASTG = -(-LASTCH // 16)
TAIL = LASTCH - (LASTG - 1) * 16
IDXV = (NCHUNK - 1) * CH + LASTG * 16 + 8
NIDX = (NW - 1) * R - ((NW - 1) * R) % 8 + IDXV


NBUF = 4


def _onehot_body(idx_hbm, zero_hbm, out_hbm, idx_v, *rest):
    bufs, sems = rest[:NBUF], rest[NBUF:]
    wid = lax.axis_index("s") * NC + lax.axis_index("c")
    base = wid * R
    abase = pl.multiple_of(base - base % 8, 8)
    off = base % 8

    pending = [pltpu.async_copy(zero_hbm, bufs[b], sems[b])
               for b in range(NBUF)]
    pltpu.sync_copy(idx_hbm.at[pl.ds(abase, IDXV)], idx_v)

    rows0 = lax.iota(jnp.int32, 16) * C
    ones = jnp.ones((16,), jnp.int32)
    zeros = jnp.zeros((16,), jnp.int32)
    tailmask = lax.iota(jnp.int32, 16) < TAIL

    for k in range(NCHUNK):
        b = k % NBUF
        last = k == NCHUNK - 1
        pending[b].wait()
        for g in range(G):
            rows = rows0 + (g * 16 * C)
            if k >= NBUF:
                old = idx_v[pl.ds(off + (k - NBUF) * CH + g * 16, 16)]
                plsc.store_scatter(bufs[b], [rows + old], zeros)
            cols = idx_v[pl.ds(off + k * CH + g * 16, 16)]
            mask = tailmask if (last and g == LASTG - 1) else None
            plsc.store_scatter(bufs[b], [rows + cols], ones, mask=mask)
        nrows = LASTCH if last else CH
        pending[b] = pltpu.async_copy(
            bufs[b].at[pl.ds(0, nrows * C)],
            out_hbm.at[pl.ds((base + k * CH) * C, nrows * C)],
            sems[b])
    for b in range(NBUF):
        pending[b].wait()


_onehot_sc = pl.kernel(
    _onehot_body,
    out_type=jax.ShapeDtypeStruct((N * C,), jnp.int32),
    mesh=plsc.VectorSubcoreMesh(core_axis_name="c", subcore_axis_name="s"),
    compiler_params=pltpu.CompilerParams(needs_layout_passes=False),
    scratch_types=(
        [pltpu.VMEM((IDXV,), jnp.int32)]
        + [pltpu.VMEM((CH * C,), jnp.int32) for _ in range(NBUF)]
        + [pltpu.SemaphoreType.DMA for _ in range(NBUF)]
    ),
)


def kernel(x, weight):
    del x
    idx = jnp.pad(weight, (0, NIDX - N))
    zero_chunk = jnp.zeros((CH * C,), jnp.int32)
    out = _onehot_sc(idx, zero_chunk)
    return out.reshape(N, C)


# speedup vs baseline: 1.8606x; 1.8606x over previous
import jax
import jax.numpy as jnp
from jax import lax
from jax.experimental import pallas as pl
from jax.experimental.pallas import tpu as pltpu
from jax.experimental.pallas import tpu_sc as plsc

N = 100000
C = 128
NC, NS = 2, 16
NW = NC * NS
R = N // NW
CH = 224
NBUF = 2
NCHUNK = -(-R // CH)
LASTCH = R - (NCHUNK - 1) * CH
G = CH // 16
LASTG = -(-LASTCH // 16)
TAIL = LASTCH - (LASTG - 1) * 16
IDXV = R + 19 // 8 * 8 + 3
IDXA = IDXV + 16


def _onehot_body(idx_hbm, out_hbm, idx_v, buf0, buf1, sem0, sem1):
    bufs = (buf0, buf1)
    sems = (sem0, sem1)
    wid = lax.axis_index("s") * NC + lax.axis_index("c")
    base = wid * R
    abase = pl.multiple_of(jnp.minimum(base - base % 8, N - IDXV), 8)
    off = base - abase

    pltpu.sync_copy(idx_hbm.at[pl.ds(abase, IDXV)], idx_v.at[pl.ds(0, IDXV)])

    rows0 = lax.iota(jnp.int32, 16) * C
    ones = jnp.ones((16,), jnp.int32)
    zeros = jnp.zeros((16,), jnp.int32)
    tailmask = lax.iota(jnp.int32, 16) < TAIL

    def zbody(i, carry):
        buf0[pl.ds(i * 16, 16)] = zeros
        buf1[pl.ds(i * 16, 16)] = zeros
        return carry
    lax.fori_loop(0, CH * C // 16, zbody, 0, unroll=8)

    def drain(b, nrows=CH):
        pltpu.make_async_copy(
            bufs[b].at[pl.ds(0, nrows * C)],
            out_hbm.at[pl.ds(0, nrows * C)], sems[b]).wait()

    def do_chunk(k, b, clear_k=None, last=False):
        for g in range(G):
            rows = rows0 + (g * 16 * C)
            if clear_k is not None:
                old = idx_v[pl.ds(off + clear_k * CH + g * 16, 16)]
                plsc.store_scatter(bufs[b], [rows + old], zeros)
            cols = idx_v[pl.ds(off + k * CH + g * 16, 16)]
            mask = tailmask if (last and g == LASTG - 1) else None
            plsc.store_scatter(bufs[b], [rows + cols], ones, mask=mask)
        nrows = LASTCH if last else CH
        pltpu.async_copy(
            bufs[b].at[pl.ds(0, nrows * C)],
            out_hbm.at[pl.ds((base + k * CH) * C, nrows * C)],
            sems[b])

    do_chunk(0, 0)
    do_chunk(1, 1)

    def loop_body(j, carry):
        k = 2 + NBUF * j
        for b in range(NBUF):
            drain(b)
            do_chunk(k + b, b, clear_k=k + b - NBUF)
        return carry
    lax.fori_loop(0, (NCHUNK - 2 - NBUF) // NBUF, loop_body, 0)

    drain(0)
    do_chunk(NCHUNK - 2, 0, clear_k=NCHUNK - 2 - NBUF)
    drain(1)
    do_chunk(NCHUNK - 1, 1, clear_k=NCHUNK - 1 - NBUF, last=True)
    drain(0)
    drain(1, nrows=LASTCH)


_onehot_sc = pl.kernel(
    _onehot_body,
    out_type=jax.ShapeDtypeStruct((N * C,), jnp.int32),
    mesh=plsc.VectorSubcoreMesh(core_axis_name="c", subcore_axis_name="s"),
    compiler_params=pltpu.CompilerParams(needs_layout_passes=False),
    scratch_types=[
        pltpu.VMEM((IDXA,), jnp.int32),
        pltpu.VMEM((CH * C,), jnp.int32),
        pltpu.VMEM((CH * C,), jnp.int32),
        pltpu.SemaphoreType.DMA,
        pltpu.SemaphoreType.DMA,
    ],
)


def kernel(x, weight):
    del x
    return _onehot_sc(weight).reshape(N, C)
